# baseline (speedup 1.0000x reference)
import functools
import jax
import jax.numpy as jnp
from jax import lax
from jax.experimental import pallas as pl
from jax.experimental.pallas import tpu as pltpu
from jax.experimental.pallas import tpu_sc as plsc

NC, NS, L = 2, 16, 16
NW = NC * NS
NEG = -1.0e30


def _node_body(v_ref, wt_ref, w12_ref, b2_ref, hv_ref, asr_ref):
    hv = jnp.dot(v_ref[...], wt_ref[...], preferred_element_type=jnp.float32)
    hv_ref[...] = hv
    asr_ref[...] = (
        jnp.dot(hv, w12_ref[...], preferred_element_type=jnp.float32)
        + b2_ref[...]
    )


def _combine_body(hv_ref, bins_ref, mx_ref, out_ref):
    mx = mx_ref[...]
    m_all = jnp.max(mx)
    scale = jnp.exp(mx[:, 0:1] - m_all)
    denom = lax.dot_general(
        bins_ref[...], scale,
        dimension_numbers=(((0,), (0,)), ((), ())),
        preferred_element_type=jnp.float32,
    )
    out_ref[...] = hv_ref[...] * (denom / (denom + 1e-8))


def _make_sc_kernel(n_pad, d_feat, n_edges):
    mesh = plsc.VectorSubcoreMesh(core_axis_name="c", subcore_axis_name="s")
    rows = n_edges // NW
    epw = ((rows + L - 1) // L) * L
    nseg = d_feat // L
    cs = 304
    chunks = [cs] * (rows // cs) + [rows % cs]
    n_half = chunks[-1] % L

    @functools.partial(
        pl.kernel,
        mesh=mesh,
        compiler_params=pltpu.CompilerParams(needs_layout_passes=False),
        out_type=(
            jax.ShapeDtypeStruct((NW, n_pad), jnp.float32),
            jax.ShapeDtypeStruct((NW, L), jnp.float32),
        ),
        scratch_types=[
            pltpu.VMEM((2 * n_pad,), jnp.float32),
            pltpu.VMEM((d_feat,), jnp.float32),
            pltpu.VMEM((2 * rows + L,), jnp.int32),
            pltpu.VMEM((epw,), jnp.float32),
            pltpu.VMEM((n_pad,), jnp.float32),
            pltpu.VMEM((L,), jnp.float32),
            pltpu.VMEM((cs, d_feat), jnp.float32),
            pltpu.VMEM((cs, d_feat), jnp.float32),
            pltpu.SemaphoreType.DMA,
            pltpu.SemaphoreType.DMA,
        ],
    )
    def sc_kernel(tab_hbm, w3_hbm, ed_hbm, e_hbm,
                  bins_out, mx_out,
                  tab_v, w3_v, ed_v, logit_v, bins_v, mx_v, eb0, eb1,
                  sem0, sem1):
        wid = lax.axis_index("s") * NC + lax.axis_index("c")
        rbase = wid * rows
        lanes = lax.iota(jnp.int32, L)
        zeros16 = jnp.zeros((L,), jnp.float32)

        ed_v[pl.ds(2 * rows, L)] = lanes * 0
        pltpu.sync_copy(tab_hbm, tab_v)
        pltpu.sync_copy(w3_hbm, w3_v)
        pltpu.sync_copy(ed_hbm.at[pl.ds(2 * rbase, 2 * rows)],
                        ed_v.at[pl.ds(0, 2 * rows)])

        def zero_body(i, carry):
            bins_v[pl.ds(i * L, L)] = zeros16
            return carry
        lax.fori_loop(0, n_pad // L, zero_body, 0)

        ebufs = (eb0, eb1)
        sems = (sem0, sem1)
        descs = [None, None]
        offs = [0]
        for csz in chunks:
            offs.append(offs[-1] + csz)
        for k in range(2):
            descs[k] = pltpu.async_copy(
                e_hbm.at[pl.ds(rbase + offs[k], chunks[k])],
                ebufs[k].at[pl.ds(0, chunks[k])], sems[k])

        def group_logits(eb, goff, g):
            w3s = [w3_v[pl.ds(s * L, L)] for s in range(nseg)]

            def row_quad(q, ae):
                row0 = g * L + q * 4
                for r in range(4):
                    acc = zeros16
                    for s in range(nseg):
                        acc = acc + eb[row0 + r, pl.ds(s * L, L)] * w3s[s]
                    ae = jnp.where(lanes == q * 4 + r, jnp.sum(acc), ae)
                return ae
            ae = lax.fori_loop(0, L // 4, row_quad, zeros16)
            eoff = goff + g * L
            pidx = lanes * 2 + eoff * 2
            si = plsc.load_gather(ed_v, [pidx])
            di = plsc.load_gather(ed_v, [pidx + 1])
            a = plsc.load_gather(tab_v, [si * 2])
            rr = plsc.load_gather(tab_v, [di * 2 + 1])
            lg = a + rr + ae
            lg = jnp.maximum(lg, lg * 0.2)
            return eoff, lg

        m = jnp.full((L,), NEG, jnp.float32)
        for k, csz in enumerate(chunks):
            descs[k % 2].wait()
            nxt = k + 2
            if nxt < len(chunks):
                descs[nxt % 2] = pltpu.async_copy(
                    e_hbm.at[pl.ds(rbase + offs[nxt], chunks[nxt])],
                    ebufs[nxt % 2].at[pl.ds(0, chunks[nxt])], sems[nxt % 2])
            eb = ebufs[k % 2]
            goff = offs[k]

            def chunk_body(g, mm, eb=eb, goff=goff):
                eoff, lg = group_logits(eb, goff, g)
                logit_v[pl.ds(eoff, L)] = lg
                return jnp.maximum(mm, lg)
            m = lax.fori_loop(0, csz // L, chunk_body, m)
            if csz % L:
                eoff, lg = group_logits(eb, goff, csz // L)
                lg = jnp.where(lanes < n_half, lg, NEG)
                logit_v[pl.ds(eoff, L)] = lg
                m = jnp.maximum(m, lg)

        m_loc = jnp.max(m)
        mx_v[...] = zeros16 + m_loc

        def accum_body(i, carry):
            att = jnp.exp(logit_v[pl.ds(i * L, L)] - m_loc)
            si = plsc.load_gather(ed_v, [lanes * 2 + i * 2 * L])
            plsc.addupdate_scatter(bins_v, [si], att)
            return carry
        lax.fori_loop(0, epw // L, accum_body, 0)

        pltpu.sync_copy(bins_v, bins_out.at[wid])
        pltpu.sync_copy(mx_v, mx_out.at[wid])

    return sc_kernel


def kernel(V, E, edges, W_f, W_a, b_a):
    B, n_nodes, d_feat = V.shape
    n_edges = edges.shape[1]
    d_out = W_f.shape[0]

    blk = 1024
    n_pad = ((n_nodes + blk - 1) // blk) * blk

    v2 = V[0]
    e2 = E[0]
    w1 = W_a[0, :d_out]
    w2 = W_a[0, d_out:2 * d_out]
    w3 = W_a[0, 2 * d_out:]
    w12 = jnp.stack([w1, w2], axis=1)
    b2 = jnp.stack([b_a[0], jnp.float32(0.0)])[None]

    hv, asr = pl.pallas_call(
        _node_body,
        grid=(n_pad // blk,),
        in_specs=[
            pl.BlockSpec((blk, d_feat), lambda i: (i, 0)),
            pl.BlockSpec((d_feat, d_out), lambda i: (0, 0)),
            pl.BlockSpec((d_out, 2), lambda i: (0, 0)),
            pl.BlockSpec((1, 2), lambda i: (0, 0)),
        ],
        out_specs=[
            pl.BlockSpec((blk, d_out), lambda i: (i, 0)),
            pl.BlockSpec((blk, 2), lambda i: (i, 0)),
        ],
        out_shape=[
            jax.ShapeDtypeStruct((n_pad, d_out), jnp.float32),
            jax.ShapeDtypeStruct((n_pad, 2), jnp.float32),
        ],
    )(v2, W_f.T, w12, b2)

    bins, mx = _make_sc_kernel(n_pad, d_feat, n_edges)(
        asr.reshape(-1), w3, edges[0].reshape(-1), e2)

    h_full = pl.pallas_call(
        _combine_body,
        grid=(n_pad // blk,),
        in_specs=[
            pl.BlockSpec((blk, d_out), lambda i: (i, 0)),
            pl.BlockSpec((NW, blk), lambda i: (0, i)),
            pl.BlockSpec((NW, L), lambda i: (0, 0)),
        ],
        out_specs=pl.BlockSpec((blk, d_out), lambda i: (i, 0)),
        out_shape=jax.ShapeDtypeStruct((n_pad, d_out), jnp.float32),
    )(hv, bins, mx)

    return h_full[:n_nodes][None]


# speedup vs baseline: 1.9284x; 1.9284x over previous
import functools
import jax
import jax.numpy as jnp
from jax import lax
from jax.experimental import pallas as pl
from jax.experimental.pallas import tpu as pltpu
from jax.experimental.pallas import tpu_sc as plsc

NC, NS, L = 2, 16, 16
NW = NC * NS
NEG = -1.0e30


def _node_body(v_ref, wt_ref, w12_ref, b2_ref, hv_ref, as_ref, ar_ref):
    hv = jnp.dot(v_ref[0], wt_ref[...], preferred_element_type=jnp.float32)
    hv_ref[...] = hv
    asr = (
        jnp.dot(hv, w12_ref[...], preferred_element_type=jnp.float32)
        + b2_ref[...]
    )
    as_ref[...] = asr[:, 0]
    ar_ref[...] = asr[:, 1]


def _combine_body(hv_ref, bins_ref, mx_ref, out_ref):
    mx = mx_ref[...]
    m_all = jnp.max(mx)
    scale = jnp.exp(mx[:, 0:1] - m_all)
    denom = lax.dot_general(
        bins_ref[...], scale,
        dimension_numbers=(((0,), (0,)), ((), ())),
        preferred_element_type=jnp.float32,
    )
    n_nodes = out_ref.shape[1]
    fac = denom[:n_nodes] / (denom[:n_nodes] + 1e-8)
    out_ref[...] = (hv_ref[:n_nodes] * fac)[None]


def _make_sc_kernel(n_pad, d_feat, n_edges):
    mesh = plsc.VectorSubcoreMesh(core_axis_name="c", subcore_axis_name="s")
    rows = n_edges // NW
    epw = ((rows + L - 1) // L) * L
    nseg = d_feat // L
    cs = 304
    chunks = [cs] * (rows // cs) + [rows % cs]
    n_half = chunks[-1] % L

    @functools.partial(
        pl.kernel,
        mesh=mesh,
        compiler_params=pltpu.CompilerParams(needs_layout_passes=False),
        out_type=(
            jax.ShapeDtypeStruct((NW, n_pad), jnp.float32),
            jax.ShapeDtypeStruct((NW, L), jnp.float32),
        ),
        scratch_types=[
            pltpu.VMEM((n_pad,), jnp.float32),
            pltpu.VMEM((n_pad,), jnp.float32),
            pltpu.VMEM((d_feat,), jnp.float32),
            pltpu.VMEM((epw,), jnp.int32),
            pltpu.VMEM((epw,), jnp.int32),
            pltpu.VMEM((epw,), jnp.float32),
            pltpu.VMEM((n_pad,), jnp.float32),
            pltpu.VMEM((L,), jnp.float32),
            pltpu.VMEM((cs, d_feat), jnp.float32),
            pltpu.VMEM((cs, d_feat), jnp.float32),
            pltpu.SemaphoreType.DMA,
            pltpu.SemaphoreType.DMA,
        ],
    )
    def sc_kernel(as_hbm, ar_hbm, w3_hbm, src_hbm, dst_hbm, e_hbm,
                  bins_out, mx_out,
                  as_v, ar_v, w3_v, src_v, dst_v, logit_v, bins_v, mx_v,
                  eb0, eb1, sem0, sem1):
        wid = lax.axis_index("s") * NC + lax.axis_index("c")
        rbase = wid * rows
        lanes = lax.iota(jnp.int32, L)
        zeros16 = jnp.zeros((L,), jnp.float32)
        izeros16 = lanes * 0

        src_v[pl.ds(epw - L, L)] = izeros16
        dst_v[pl.ds(epw - L, L)] = izeros16
        pltpu.sync_copy(as_hbm, as_v)
        pltpu.sync_copy(ar_hbm, ar_v)
        pltpu.sync_copy(w3_hbm, w3_v)
        pltpu.sync_copy(src_hbm.at[pl.ds(rbase, rows)],
                        src_v.at[pl.ds(0, rows)])
        pltpu.sync_copy(dst_hbm.at[pl.ds(rbase, rows)],
                        dst_v.at[pl.ds(0, rows)])

        def zero_body(i, carry):
            bins_v[pl.ds(i * L, L)] = zeros16
            return carry
        lax.fori_loop(0, n_pad // L, zero_body, 0)

        ebufs = (eb0, eb1)
        sems = (sem0, sem1)
        descs = [None, None]
        offs = [0]
        for csz in chunks:
            offs.append(offs[-1] + csz)
        for k in range(2):
            descs[k] = pltpu.async_copy(
                e_hbm.at[0, pl.ds(rbase + offs[k], chunks[k])],
                ebufs[k].at[pl.ds(0, chunks[k])], sems[k])

        def group_logits(eb, goff, g):
            w3s = [w3_v[pl.ds(s * L, L)] for s in range(nseg)]

            def row_quad(q, ae):
                row0 = g * L + q * 4
                for r in range(4):
                    acc = zeros16
                    for s in range(nseg):
                        acc = acc + eb[row0 + r, pl.ds(s * L, L)] * w3s[s]
                    ae = jnp.where(lanes == q * 4 + r, jnp.sum(acc), ae)
                return ae
            ae = lax.fori_loop(0, L // 4, row_quad, zeros16)
            eoff = goff + g * L
            eidx = lanes + eoff
            si = plsc.load_gather(src_v, [eidx])
            di = plsc.load_gather(dst_v, [eidx])
            a = plsc.load_gather(as_v, [si])
            rr = plsc.load_gather(ar_v, [di])
            lg = a + rr + ae
            lg = jnp.maximum(lg, lg * 0.2)
            return eoff, lg

        m = jnp.full((L,), NEG, jnp.float32)
        for k, csz in enumerate(chunks):
            descs[k % 2].wait()
            nxt = k + 2
            if nxt < len(chunks):
                descs[nxt % 2] = pltpu.async_copy(
                    e_hbm.at[0, pl.ds(rbase + offs[nxt], chunks[nxt])],
                    ebufs[nxt % 2].at[pl.ds(0, chunks[nxt])], sems[nxt % 2])
            eb = ebufs[k % 2]
            goff = offs[k]

            def chunk_body(g, mm, eb=eb, goff=goff):
                eoff, lg = group_logits(eb, goff, g)
                logit_v[pl.ds(eoff, L)] = lg
                return jnp.maximum(mm, lg)
            m = lax.fori_loop(0, csz // L, chunk_body, m)
            if csz % L:
                eoff, lg = group_logits(eb, goff, csz // L)
                lg = jnp.where(lanes < n_half, lg, NEG)
                logit_v[pl.ds(eoff, L)] = lg
                m = jnp.maximum(m, lg)

        m_loc = jnp.max(m)
        mx_v[...] = zeros16 + m_loc

        def accum_body(i, carry):
            att = jnp.exp(logit_v[pl.ds(i * L, L)] - m_loc)
            si = src_v[pl.ds(i * L, L)]
            plsc.addupdate_scatter(bins_v, [si], att)
            return carry
        lax.fori_loop(0, epw // L, accum_body, 0)

        pltpu.sync_copy(bins_v, bins_out.at[wid])
        pltpu.sync_copy(mx_v, mx_out.at[wid])

    return sc_kernel


def kernel(V, E, edges, W_f, W_a, b_a):
    B, n_nodes, d_feat = V.shape
    n_edges = edges.shape[1]
    d_out = W_f.shape[0]

    blk = 1024
    n_pad = ((n_nodes + blk - 1) // blk) * blk

    w1 = W_a[0, :d_out]
    w2 = W_a[0, d_out:2 * d_out]
    w3 = W_a[0, 2 * d_out:]
    w12 = jnp.stack([w1, w2], axis=1)
    b2 = jnp.stack([b_a[0], jnp.float32(0.0)])[None]

    hv, a_s, a_r = pl.pallas_call(
        _node_body,
        grid=(n_pad // blk,),
        in_specs=[
            pl.BlockSpec((1, blk, d_feat), lambda i: (0, i, 0)),
            pl.BlockSpec((d_feat, d_out), lambda i: (0, 0)),
            pl.BlockSpec((d_out, 2), lambda i: (0, 0)),
            pl.BlockSpec((1, 2), lambda i: (0, 0)),
        ],
        out_specs=[
            pl.BlockSpec((blk, d_out), lambda i: (i, 0)),
            pl.BlockSpec((blk,), lambda i: (i,)),
            pl.BlockSpec((blk,), lambda i: (i,)),
        ],
        out_shape=[
            jax.ShapeDtypeStruct((n_pad, d_out), jnp.float32),
            jax.ShapeDtypeStruct((n_pad,), jnp.float32),
            jax.ShapeDtypeStruct((n_pad,), jnp.float32),
        ],
    )(V, W_f.T, w12, b2)

    bins, mx = _make_sc_kernel(n_pad, d_feat, n_edges)(
        a_s, a_r, w3, edges[0, :, 0], edges[0, :, 1], E)

    h = pl.pallas_call(
        _combine_body,
        in_specs=[
            pl.BlockSpec((n_pad, d_out), lambda: (0, 0)),
            pl.BlockSpec((NW, n_pad), lambda: (0, 0)),
            pl.BlockSpec((NW, L), lambda: (0, 0)),
        ],
        out_specs=pl.BlockSpec((1, n_nodes, d_out), lambda: (0, 0, 0)),
        out_shape=jax.ShapeDtypeStruct((1, n_nodes, d_out), jnp.float32),
    )(hv, bins, mx)

    return h


# speedup vs baseline: 1.9526x; 1.0125x over previous
import functools
import jax
import jax.numpy as jnp
from jax import lax
from jax.experimental import pallas as pl
from jax.experimental.pallas import tpu as pltpu
from jax.experimental.pallas import tpu_sc as plsc

NC, NS, L = 2, 16, 16
NW = NC * NS
NEG = -1.0e30


def _node_body(v_ref, wt_ref, w12_ref, b2_ref, hv_ref, as_ref, ar_ref):
    hv = jnp.dot(v_ref[0], wt_ref[...], preferred_element_type=jnp.float32)
    hv_ref[...] = hv
    asr = (
        jnp.dot(hv, w12_ref[...], preferred_element_type=jnp.float32)
        + b2_ref[...]
    )
    as_ref[...] = asr[:, 0]
    ar_ref[...] = asr[:, 1]


def _combine_body(hv_ref, bins_ref, mx_ref, out_ref):
    mx = mx_ref[...]
    m_all = jnp.max(mx)
    scale = jnp.exp(mx[:, 0:1] - m_all)
    denom = lax.dot_general(
        bins_ref[...], scale,
        dimension_numbers=(((0,), (0,)), ((), ())),
        preferred_element_type=jnp.float32,
    )
    n_nodes = out_ref.shape[1]
    fac = denom[:n_nodes] / (denom[:n_nodes] + 1e-8)
    out_ref[...] = (hv_ref[:n_nodes] * fac)[None]


def _make_sc_kernel(n_pad, d_feat, n_edges):
    mesh = plsc.VectorSubcoreMesh(core_axis_name="c", subcore_axis_name="s")
    rows = n_edges // NW
    epw = ((rows + L - 1) // L) * L
    nseg = d_feat // L
    cs = 304
    chunks = [cs] * (rows // cs) + [rows % cs]
    n_half = chunks[-1] % L

    @functools.partial(
        pl.kernel,
        mesh=mesh,
        compiler_params=pltpu.CompilerParams(needs_layout_passes=False),
        out_type=(
            jax.ShapeDtypeStruct((NW, n_pad), jnp.float32),
            jax.ShapeDtypeStruct((NW, L), jnp.float32),
        ),
        scratch_types=[
            pltpu.VMEM((n_pad,), jnp.float32),
            pltpu.VMEM((n_pad,), jnp.float32),
            pltpu.VMEM((d_feat,), jnp.float32),
            pltpu.VMEM((epw,), jnp.int32),
            pltpu.VMEM((epw,), jnp.int32),
            pltpu.VMEM((epw,), jnp.float32),
            pltpu.VMEM((n_pad,), jnp.float32),
            pltpu.VMEM((L,), jnp.float32),
            pltpu.VMEM((cs, d_feat), jnp.float32),
            pltpu.VMEM((cs, d_feat), jnp.float32),
            pltpu.SemaphoreType.DMA,
            pltpu.SemaphoreType.DMA,
        ],
    )
    def sc_kernel(as_hbm, ar_hbm, w3_hbm, src_hbm, dst_hbm, e_hbm,
                  bins_out, mx_out,
                  as_v, ar_v, w3_v, src_v, dst_v, logit_v, bins_v, mx_v,
                  eb0, eb1, sem0, sem1):
        wid = lax.axis_index("s") * NC + lax.axis_index("c")
        rbase = wid * rows
        lanes = lax.iota(jnp.int32, L)
        zeros16 = jnp.zeros((L,), jnp.float32)
        izeros16 = lanes * 0

        src_v[pl.ds(epw - L, L)] = izeros16
        dst_v[pl.ds(epw - L, L)] = izeros16
        pltpu.sync_copy(as_hbm, as_v)
        pltpu.sync_copy(ar_hbm, ar_v)
        pltpu.sync_copy(w3_hbm, w3_v)
        pltpu.sync_copy(src_hbm.at[pl.ds(rbase, rows)],
                        src_v.at[pl.ds(0, rows)])
        pltpu.sync_copy(dst_hbm.at[pl.ds(rbase, rows)],
                        dst_v.at[pl.ds(0, rows)])

        def zero_body(i, carry):
            bins_v[pl.ds(i * L, L)] = zeros16
            return carry
        lax.fori_loop(0, n_pad // L, zero_body, 0)

        ebufs = (eb0, eb1)
        sems = (sem0, sem1)
        descs = [None, None]
        offs = [0]
        for csz in chunks:
            offs.append(offs[-1] + csz)
        for k in range(2):
            descs[k] = pltpu.async_copy(
                e_hbm.at[0, pl.ds(rbase + offs[k], chunks[k])],
                ebufs[k].at[pl.ds(0, chunks[k])], sems[k])

        def group_logits(eb, goff, g):
            w3s = [w3_v[pl.ds(s * L, L)] for s in range(nseg)]

            def row_quad(q, ae):
                row0 = g * L + q * 4
                for r in range(4):
                    acc0 = eb[row0 + r, pl.ds(0, L)] * w3s[0]
                    acc1 = eb[row0 + r, pl.ds(L, L)] * w3s[1]
                    for s in range(2, nseg, 2):
                        acc0 = acc0 + eb[row0 + r, pl.ds(s * L, L)] * w3s[s]
                        acc1 = acc1 + eb[row0 + r, pl.ds((s + 1) * L, L)] * w3s[s + 1]
                    ae = jnp.where(lanes == q * 4 + r, jnp.sum(acc0 + acc1), ae)
                return ae
            ae = lax.fori_loop(0, L // 4, row_quad, zeros16)
            eoff = goff + g * L
            eidx = lanes + eoff
            si = plsc.load_gather(src_v, [eidx])
            di = plsc.load_gather(dst_v, [eidx])
            a = plsc.load_gather(as_v, [si])
            rr = plsc.load_gather(ar_v, [di])
            lg = a + rr + ae
            lg = jnp.maximum(lg, lg * 0.2)
            return eoff, lg

        m = jnp.full((L,), NEG, jnp.float32)
        for k, csz in enumerate(chunks):
            descs[k % 2].wait()
            nxt = k + 2
            if nxt < len(chunks):
                descs[nxt % 2] = pltpu.async_copy(
                    e_hbm.at[0, pl.ds(rbase + offs[nxt], chunks[nxt])],
                    ebufs[nxt % 2].at[pl.ds(0, chunks[nxt])], sems[nxt % 2])
            eb = ebufs[k % 2]
            goff = offs[k]

            def chunk_body(g, mm, eb=eb, goff=goff):
                eoff, lg = group_logits(eb, goff, g)
                logit_v[pl.ds(eoff, L)] = lg
                return jnp.maximum(mm, lg)
            m = lax.fori_loop(0, csz // L, chunk_body, m)
            if csz % L:
                eoff, lg = group_logits(eb, goff, csz // L)
                lg = jnp.where(lanes < n_half, lg, NEG)
                logit_v[pl.ds(eoff, L)] = lg
                m = jnp.maximum(m, lg)

        m_loc = jnp.max(m)
        mx_v[...] = zeros16 + m_loc

        def accum_body(i, carry):
            att = jnp.exp(logit_v[pl.ds(i * L, L)] - m_loc)
            si = src_v[pl.ds(i * L, L)]
            plsc.addupdate_scatter(bins_v, [si], att)
            return carry
        lax.fori_loop(0, epw // L, accum_body, 0)

        pltpu.sync_copy(bins_v, bins_out.at[wid])
        pltpu.sync_copy(mx_v, mx_out.at[wid])

    return sc_kernel


def kernel(V, E, edges, W_f, W_a, b_a):
    B, n_nodes, d_feat = V.shape
    n_edges = edges.shape[1]
    d_out = W_f.shape[0]

    blk = 1024
    n_pad = ((n_nodes + blk - 1) // blk) * blk

    w1 = W_a[0, :d_out]
    w2 = W_a[0, d_out:2 * d_out]
    w3 = W_a[0, 2 * d_out:]
    w12 = jnp.stack([w1, w2], axis=1)
    b2 = jnp.stack([b_a[0], jnp.float32(0.0)])[None]

    hv, a_s, a_r = pl.pallas_call(
        _node_body,
        grid=(n_pad // blk,),
        in_specs=[
            pl.BlockSpec((1, blk, d_feat), lambda i: (0, i, 0)),
            pl.BlockSpec((d_feat, d_out), lambda i: (0, 0)),
            pl.BlockSpec((d_out, 2), lambda i: (0, 0)),
            pl.BlockSpec((1, 2), lambda i: (0, 0)),
        ],
        out_specs=[
            pl.BlockSpec((blk, d_out), lambda i: (i, 0)),
            pl.BlockSpec((blk,), lambda i: (i,)),
            pl.BlockSpec((blk,), lambda i: (i,)),
        ],
        out_shape=[
            jax.ShapeDtypeStruct((n_pad, d_out), jnp.float32),
            jax.ShapeDtypeStruct((n_pad,), jnp.float32),
            jax.ShapeDtypeStruct((n_pad,), jnp.float32),
        ],
    )(V, W_f.T, w12, b2)

    bins, mx = _make_sc_kernel(n_pad, d_feat, n_edges)(
        a_s, a_r, w3, edges[0, :, 0], edges[0, :, 1], E)

    h = pl.pallas_call(
        _combine_body,
        in_specs=[
            pl.BlockSpec((n_pad, d_out), lambda: (0, 0)),
            pl.BlockSpec((NW, n_pad), lambda: (0, 0)),
            pl.BlockSpec((NW, L), lambda: (0, 0)),
        ],
        out_specs=pl.BlockSpec((1, n_nodes, d_out), lambda: (0, 0, 0)),
        out_shape=jax.ShapeDtypeStruct((1, n_nodes, d_out), jnp.float32),
    )(hv, bins, mx)

    return h
